# baseline (speedup 1.0000x reference)
import functools

import jax
import jax.numpy as jnp
from jax import lax
from jax.experimental import pallas as pl
from jax.experimental.pallas import tpu as pltpu
from jax.experimental.pallas import tpu_sc as plsc

NC = 2
NS = 16
NW = NC * NS
CH = 128
HEADS = 4
XAUG_D = 80
SI_D = 16


def _prep_body(xtx_ref, w1_ref, b1_ref, w2_ref, b2_ref, xaug_ref, si_ref):
    x = xtx_ref[...]
    xaug_ref[...] = (
        jnp.dot(x, w1_ref[...], preferred_element_type=jnp.float32) + b1_ref[...]
    )
    si_ref[...] = (
        jnp.dot(x, w2_ref[...], preferred_element_type=jnp.float32) + b2_ref[...]
    )


def _prep(x_tx, W1, b1, W2, b2):
    n = x_tx.shape[0]
    blk = 1000
    return pl.pallas_call(
        _prep_body,
        grid=(n // blk,),
        in_specs=[
            pl.BlockSpec((blk, 32), lambda i: (i, 0)),
            pl.BlockSpec((32, XAUG_D), lambda i: (0, 0)),
            pl.BlockSpec((1, XAUG_D), lambda i: (0, 0)),
            pl.BlockSpec((32, SI_D), lambda i: (0, 0)),
            pl.BlockSpec((1, SI_D), lambda i: (0, 0)),
        ],
        out_specs=[
            pl.BlockSpec((blk, XAUG_D), lambda i: (i, 0)),
            pl.BlockSpec((blk, SI_D), lambda i: (i, 0)),
        ],
        out_shape=[
            jax.ShapeDtypeStruct((n, XAUG_D), jnp.float32),
            jax.ShapeDtypeStruct((n, SI_D), jnp.float32),
        ],
    )(x_tx, W1, b1, W2, b2)


NB = 4
SBC = 28
NSB = 7


def _make_edge_kernel(n_nodes, n_edges):
    n_chunks = n_edges // CH
    cpw = SBC * NSB
    zrows = 200
    n_rchunks = n_nodes // zrows
    mesh = plsc.VectorSubcoreMesh(core_axis_name="c", subcore_axis_name="s")

    @functools.partial(
        pl.kernel,
        mesh=mesh,
        out_type=jax.ShapeDtypeStruct((NC * n_nodes, 16), jnp.float32),
        scratch_types=[
            pltpu.VMEM((SBC, 3, CH), jnp.int32),
            pltpu.VMEM((NB, CH, XAUG_D), jnp.float32),
            pltpu.VMEM((NB, CH, SI_D), jnp.float32),
            pltpu.VMEM((NB, CH, 16), jnp.float32),
            pltpu.VMEM((zrows, 16), jnp.float32),
            pltpu.VMEM((16,), jnp.float32),
            pltpu.VMEM_SHARED((n_nodes + 8, 16), jnp.float32),
            pltpu.SemaphoreType.DMA,
            pltpu.SemaphoreType.DMA,
            pltpu.SemaphoreType.DMA,
            pltpu.SemaphoreType.DMA,
            pltpu.SemaphoreType.DMA,
            pltpu.SemaphoreType.DMA,
            pltpu.SemaphoreType.DMA,
            pltpu.SemaphoreType.DMA,
            pltpu.SemaphoreType.DMA,
        ],
        compiler_params=pltpu.CompilerParams(use_tc_tiling_on_sc=False,
                                             needs_layout_passes=False),
    )
    def edge_kernel(pidx_hbm, negbeta_hbm, xaug_hbm, si_hbm, out_hbm,
                    ibuf, xrows, sirows, msg, zbuf, nb_v, acc,
                    sem_i, sem_g0, sem_g1, sem_g2, sem_g3,
                    sem_s0, sem_s1, sem_s2, sem_s3):
        sem_g = [sem_g0, sem_g1, sem_g2, sem_g3]
        sem_s = [sem_s0, sem_s1, sem_s2, sem_s3]
        cid = lax.axis_index("c")
        sid = lax.axis_index("s")
        wid = sid * NC + cid
        start = wid * cpw

        pltpu.async_copy(pidx_hbm.at[pl.ds(start, SBC)], ibuf, sem_i)
        pltpu.sync_copy(negbeta_hbm, nb_v)

        def zrow_body(i, carry):
            zbuf[i, pl.ds(0, 16)] = jnp.zeros((16,), jnp.float32)
            return carry

        lax.fori_loop(0, zrows, zrow_body, 0)
        n_my_rchunks = (n_rchunks - sid + NS - 1) // NS

        def zchunk_body(j, carry):
            r0 = (sid + j * NS) * zrows
            pltpu.sync_copy(zbuf, acc.at[pl.ds(r0, zrows)])
            return carry

        lax.fori_loop(0, n_my_rchunks, zchunk_body, 0)
        plsc.subcore_barrier()

        nbvec = nb_v[pl.ds(0, 16)]
        lanes = lax.iota(jnp.int32, 16)
        zl = lanes * 0

        def issue_gather(j, b):
            pltpu.async_copy(xaug_hbm.at[ibuf.at[j, 0]], xrows.at[b],
                             sem_g[b])
            pltpu.async_copy(si_hbm.at[ibuf.at[j, 1]], sirows.at[b],
                             sem_g[b])

        def wait_gather(j, b):
            pltpu.make_async_copy(xaug_hbm.at[ibuf.at[j, 0]],
                                  xrows.at[b], sem_g[b]).wait()
            pltpu.make_async_copy(si_hbm.at[ibuf.at[j, 1]],
                                  sirows.at[b], sem_g[b]).wait()

        def compute_chunk(j, b):
            def group_body(g, gcarry):
                e0 = g * 16
                eidx = lanes + e0
                t = plsc.bitcast(ibuf[j, 2, pl.ds(e0, 16)], jnp.float32)
                tw = jnp.exp(t * nbvec)
                alphas = []
                for h in range(HEADS):
                    col = zl + h
                    si_h = plsc.load_gather(sirows.at[b], [eidx, col])
                    sj_h = plsc.load_gather(xrows.at[b], [eidx, col + 64])
                    a = si_h + sj_h
                    a = jnp.maximum(a, 0.2 * a) * tw
                    alphas.append(1.0 / (1.0 + jnp.exp(-a)))
                for lane in range(16):
                    e = e0 + lane
                    m = (xrows[b, e, pl.ds(0, 16)] * alphas[0][lane]
                         + xrows[b, e, pl.ds(16, 16)] * alphas[1][lane]
                         + xrows[b, e, pl.ds(32, 16)] * alphas[2][lane]
                         + xrows[b, e, pl.ds(48, 16)] * alphas[3][lane])
                    msg[b, e, pl.ds(0, 16)] = m
                return gcarry

            lax.fori_loop(0, CH // 16, group_body, 0)

        def wait_scatter(b):
            pltpu.make_async_copy(msg.at[b], acc.at[ibuf.at[b, 1]],
                                  sem_s[b]).wait()

        def sb_body(s, carry):
            @pl.when(s > 0)
            def _():
                for b in range(NB):
                    wait_scatter(b)
                pltpu.async_copy(
                    pidx_hbm.at[pl.ds(start + s * SBC, SBC)], ibuf, sem_i)

            pltpu.make_async_copy(
                pidx_hbm.at[pl.ds(start + s * SBC, SBC)], ibuf, sem_i).wait()

            for b in range(NB):
                issue_gather(b, b)

            def q_body(q, qcarry):
                for b in range(NB):
                    j = q * NB + b
                    wait_gather(j, b)

                    @pl.when(q > 0)
                    def _():
                        wait_scatter(b)

                    compute_chunk(j, b)

                    @pl.when(j + NB < SBC)
                    def _():
                        issue_gather(j + NB, b)

                    pltpu.async_copy(msg.at[b], acc.at[ibuf.at[j, 1]],
                                     sem_s[b], add=True)
                return qcarry

            lax.fori_loop(0, SBC // NB, q_body, 0)
            return carry

        lax.fori_loop(0, NSB, sb_body, 0)
        for b in range(NB):
            wait_scatter(b)

        plsc.subcore_barrier()

        def ochunk_body(j, carry):
            r0 = (sid + j * NS) * zrows
            pltpu.sync_copy(acc.at[pl.ds(r0, zrows)],
                            out_hbm.at[pl.ds(cid * n_nodes + r0, zrows)])
            return carry

        lax.fori_loop(0, n_my_rchunks, ochunk_body, 0)

    return edge_kernel


def _tail_body(p0_ref, p1_ref, wc_ref, bc_ref, out_ref):
    h = 0.25 * (p0_ref[...] + p1_ref[...])
    h = jnp.where(h > 0, h, jnp.exp(h) - 1.0)
    out_ref[...] = (
        jnp.dot(h, wc_ref[...], preferred_element_type=jnp.float32) + bc_ref[...]
    )


def _tail(partial, WcT, bc2, n_nodes):
    blk = 1000
    nb = n_nodes // blk
    out_d = WcT.shape[1]
    return pl.pallas_call(
        _tail_body,
        grid=(nb,),
        in_specs=[
            pl.BlockSpec((blk, 16), lambda i: (i, 0)),
            pl.BlockSpec((blk, 16), lambda i, nb=nb: (nb + i, 0)),
            pl.BlockSpec((16, out_d), lambda i: (0, 0)),
            pl.BlockSpec((1, out_d), lambda i: (0, 0)),
        ],
        out_specs=pl.BlockSpec((blk, out_d), lambda i: (i, 0)),
        out_shape=jax.ShapeDtypeStruct((n_nodes, out_d), jnp.float32),
    )(partial, partial, WcT, bc2)


def kernel(x_user, x_tx, edge_index, edge_time, Wu, bu, Wt, bt, Wlin, att,
           time_beta, Wc, bc):
    H = att.shape[1]
    C = att.shape[2] // 2
    n_nodes = x_tx.shape[0]
    n_edges = edge_index.shape[1]

    Wx = Wt.T @ Wlin.T
    bx = bt @ Wlin.T
    att_i = att[0, :, :C]
    att_j = att[0, :, C:]
    eye = jnp.eye(H, dtype=jnp.float32)
    A_i = (att_i[:, :, None] * eye[:, None, :]).reshape(H * C, H)
    A_j = (att_j[:, :, None] * eye[:, None, :]).reshape(H * C, H)
    W1 = jnp.concatenate([Wx, Wx @ A_j, jnp.zeros((32, XAUG_D - 68))], axis=1)
    b1 = jnp.concatenate([bx, bx @ A_j, jnp.zeros(XAUG_D - 68)])[None]
    W2 = jnp.concatenate([Wx @ A_i, jnp.zeros((32, SI_D - 4))], axis=1)
    b2 = jnp.concatenate([bx @ A_i, jnp.zeros(SI_D - 4)])[None]

    xaug, si = _prep(x_tx, W1, b1, W2, b2)

    negbeta = jnp.full((16,), -jax.nn.softplus(time_beta), dtype=jnp.float32)

    n_chunks = n_edges // CH
    tbits = jax.lax.bitcast_convert_type(edge_time, jnp.int32)
    pidx = jnp.stack(
        [edge_index[0].reshape(n_chunks, CH),
         edge_index[1].reshape(n_chunks, CH),
         tbits.reshape(n_chunks, CH)], axis=1)
    pad = NW * SBC * NSB - n_chunks
    pad_block = jnp.stack(
        [jnp.zeros((pad, CH), jnp.int32),
         jnp.full((pad, CH), n_nodes, jnp.int32),
         jnp.zeros((pad, CH), jnp.int32)], axis=1)
    pidx = jnp.concatenate([pidx, pad_block], axis=0)

    edge_kernel = _make_edge_kernel(n_nodes, n_edges)
    partial = edge_kernel(pidx, negbeta, xaug, si)

    return _tail(partial, Wc.T, bc[None], n_nodes)


# speedup vs baseline: 1.0931x; 1.0931x over previous
import functools

import jax
import jax.numpy as jnp
from jax import lax
from jax.experimental import pallas as pl
from jax.experimental.pallas import tpu as pltpu
from jax.experimental.pallas import tpu_sc as plsc

NC = 2
NS = 16
NW = NC * NS
CH = 128
HEADS = 4
XAUG_D = 80
SI_D = 16


def _prep_body(xtx_ref, w1_ref, b1_ref, w2_ref, b2_ref, xaug_ref, si_ref):
    x = xtx_ref[...]
    xaug_ref[...] = (
        jnp.dot(x, w1_ref[...], preferred_element_type=jnp.float32) + b1_ref[...]
    )
    si_ref[...] = (
        jnp.dot(x, w2_ref[...], preferred_element_type=jnp.float32) + b2_ref[...]
    )


def _prep(x_tx, W1, b1, W2, b2):
    n = x_tx.shape[0]
    blk = 5000
    return pl.pallas_call(
        _prep_body,
        grid=(n // blk,),
        in_specs=[
            pl.BlockSpec((blk, 32), lambda i: (i, 0)),
            pl.BlockSpec((32, XAUG_D), lambda i: (0, 0)),
            pl.BlockSpec((1, XAUG_D), lambda i: (0, 0)),
            pl.BlockSpec((32, SI_D), lambda i: (0, 0)),
            pl.BlockSpec((1, SI_D), lambda i: (0, 0)),
        ],
        out_specs=[
            pl.BlockSpec((blk, XAUG_D), lambda i: (i, 0)),
            pl.BlockSpec((blk, SI_D), lambda i: (i, 0)),
        ],
        out_shape=[
            jax.ShapeDtypeStruct((n, XAUG_D), jnp.float32),
            jax.ShapeDtypeStruct((n, SI_D), jnp.float32),
        ],
    )(x_tx, W1, b1, W2, b2)


NB = 4
SBC = 28
NSB = 7


def _make_edge_kernel(n_nodes, n_edges):
    n_chunks = n_edges // CH
    cpw = SBC * NSB
    zrows = 200
    n_rchunks = n_nodes // zrows
    mesh = plsc.VectorSubcoreMesh(core_axis_name="c", subcore_axis_name="s")

    @functools.partial(
        pl.kernel,
        mesh=mesh,
        out_type=jax.ShapeDtypeStruct((NC * n_nodes, 16), jnp.float32),
        scratch_types=[
            pltpu.VMEM((SBC * CH,), jnp.int32),
            pltpu.VMEM((SBC, CH), jnp.int32),
            pltpu.VMEM((SBC * CH,), jnp.int32),
            pltpu.VMEM((NB, CH, XAUG_D), jnp.float32),
            pltpu.VMEM((NB, CH, SI_D), jnp.float32),
            pltpu.VMEM((NB, CH, 16), jnp.float32),
            pltpu.VMEM((zrows, 16), jnp.float32),
            pltpu.VMEM((16,), jnp.float32),
            pltpu.VMEM_SHARED((n_nodes + 8, 16), jnp.float32),
            pltpu.SemaphoreType.DMA,
            pltpu.SemaphoreType.DMA,
            pltpu.SemaphoreType.DMA,
            pltpu.SemaphoreType.DMA,
            pltpu.SemaphoreType.DMA,
            pltpu.SemaphoreType.DMA,
            pltpu.SemaphoreType.DMA,
            pltpu.SemaphoreType.DMA,
            pltpu.SemaphoreType.DMA,
        ],
        compiler_params=pltpu.CompilerParams(use_tc_tiling_on_sc=False,
                                             needs_layout_passes=False),
    )
    def edge_kernel(src_hbm, dst_hbm, t_hbm, negbeta_hbm, xaug_hbm, si_hbm,
                    out_hbm, isrc, idst, itim, xrows, sirows, msg, zbuf,
                    nb_v, acc, sem_i, sem_g0, sem_g1, sem_g2, sem_g3,
                    sem_s0, sem_s1, sem_s2, sem_s3):
        sem_g = [sem_g0, sem_g1, sem_g2, sem_g3]
        sem_s = [sem_s0, sem_s1, sem_s2, sem_s3]
        cid = lax.axis_index("c")
        sid = lax.axis_index("s")
        wid = sid * NC + cid
        start = wid * cpw

        def issue_idx(s):
            e0 = (start + s * SBC) * CH
            pltpu.async_copy(src_hbm.at[pl.ds(e0, SBC * CH)], isrc, sem_i)
            pltpu.async_copy(dst_hbm.at[pl.ds(start + s * SBC, SBC)], idst,
                             sem_i)
            pltpu.async_copy(t_hbm.at[pl.ds(e0, SBC * CH)], itim, sem_i)

        def wait_idx(s):
            e0 = (start + s * SBC) * CH
            pltpu.make_async_copy(src_hbm.at[pl.ds(e0, SBC * CH)], isrc,
                                  sem_i).wait()
            pltpu.make_async_copy(dst_hbm.at[pl.ds(start + s * SBC, SBC)],
                                  idst, sem_i).wait()
            pltpu.make_async_copy(t_hbm.at[pl.ds(e0, SBC * CH)], itim,
                                  sem_i).wait()

        issue_idx(0)
        pltpu.sync_copy(negbeta_hbm, nb_v)

        def zrow_body(i, carry):
            zbuf[i, pl.ds(0, 16)] = jnp.zeros((16,), jnp.float32)
            return carry

        lax.fori_loop(0, zrows, zrow_body, 0)
        n_my_rchunks = (n_rchunks - sid + NS - 1) // NS

        def zchunk_body(j, carry):
            r0 = (sid + j * NS) * zrows
            pltpu.sync_copy(zbuf, acc.at[pl.ds(r0, zrows)])
            return carry

        lax.fori_loop(0, n_my_rchunks, zchunk_body, 0)
        plsc.subcore_barrier()

        nbvec = nb_v[pl.ds(0, 16)]
        lanes = lax.iota(jnp.int32, 16)
        zl = lanes * 0

        def issue_gather(j, b):
            pltpu.async_copy(xaug_hbm.at[isrc.at[pl.ds(j * CH, CH)]],
                             xrows.at[b], sem_g[b])
            pltpu.async_copy(si_hbm.at[idst.at[j]], sirows.at[b],
                             sem_g[b])

        def wait_gather(j, b):
            pltpu.make_async_copy(xaug_hbm.at[isrc.at[pl.ds(j * CH, CH)]],
                                  xrows.at[b], sem_g[b]).wait()
            pltpu.make_async_copy(si_hbm.at[idst.at[j]],
                                  sirows.at[b], sem_g[b]).wait()

        def compute_chunk(j, b):
            def group_body(g, gcarry):
                e0 = g * 16
                eidx = lanes + e0
                t = plsc.bitcast(itim[pl.ds(j * CH + e0, 16)], jnp.float32)
                tw = jnp.exp(t * nbvec)
                alphas = []
                for h in range(HEADS):
                    col = zl + h
                    si_h = plsc.load_gather(sirows.at[b], [eidx, col])
                    sj_h = plsc.load_gather(xrows.at[b], [eidx, col + 64])
                    a = si_h + sj_h
                    a = jnp.maximum(a, 0.2 * a) * tw
                    alphas.append(1.0 / (1.0 + jnp.exp(-a)))
                for lane in range(16):
                    e = e0 + lane
                    m = (xrows[b, e, pl.ds(0, 16)] * alphas[0][lane]
                         + xrows[b, e, pl.ds(16, 16)] * alphas[1][lane]
                         + xrows[b, e, pl.ds(32, 16)] * alphas[2][lane]
                         + xrows[b, e, pl.ds(48, 16)] * alphas[3][lane])
                    msg[b, e, pl.ds(0, 16)] = m
                return gcarry

            lax.fori_loop(0, CH // 16, group_body, 0)

        def wait_scatter(b):
            pltpu.make_async_copy(msg.at[b], acc.at[idst.at[b]],
                                  sem_s[b]).wait()

        def sb_body(s, carry):
            @pl.when(s > 0)
            def _():
                for b in range(NB):
                    wait_scatter(b)
                issue_idx(s)

            wait_idx(s)

            for b in range(NB):
                issue_gather(b, b)

            def q_body(q, qcarry):
                for b in range(NB):
                    j = q * NB + b
                    wait_gather(j, b)

                    @pl.when(q > 0)
                    def _():
                        wait_scatter(b)

                    compute_chunk(j, b)

                    @pl.when(j + NB < SBC)
                    def _():
                        issue_gather(j + NB, b)

                    pltpu.async_copy(msg.at[b], acc.at[idst.at[j]],
                                     sem_s[b], add=True)
                return qcarry

            lax.fori_loop(0, SBC // NB, q_body, 0)
            return carry

        lax.fori_loop(0, NSB, sb_body, 0)
        for b in range(NB):
            wait_scatter(b)

        plsc.subcore_barrier()

        def ochunk_body(j, carry):
            r0 = (sid + j * NS) * zrows
            pltpu.sync_copy(acc.at[pl.ds(r0, zrows)],
                            out_hbm.at[pl.ds(cid * n_nodes + r0, zrows)])
            return carry

        lax.fori_loop(0, n_my_rchunks, ochunk_body, 0)

    return edge_kernel


def _tail_body(p0_ref, p1_ref, wc_ref, bc_ref, out_ref):
    h = 0.25 * (p0_ref[...] + p1_ref[...])
    h = jnp.where(h > 0, h, jnp.exp(h) - 1.0)
    out_ref[...] = (
        jnp.dot(h, wc_ref[...], preferred_element_type=jnp.float32) + bc_ref[...]
    )


def _tail(partial, WcT, bc2, n_nodes):
    blk = 5000
    nb = n_nodes // blk
    out_d = WcT.shape[1]
    return pl.pallas_call(
        _tail_body,
        grid=(nb,),
        in_specs=[
            pl.BlockSpec((blk, 16), lambda i: (i, 0)),
            pl.BlockSpec((blk, 16), lambda i, nb=nb: (nb + i, 0)),
            pl.BlockSpec((16, out_d), lambda i: (0, 0)),
            pl.BlockSpec((1, out_d), lambda i: (0, 0)),
        ],
        out_specs=pl.BlockSpec((blk, out_d), lambda i: (i, 0)),
        out_shape=jax.ShapeDtypeStruct((n_nodes, out_d), jnp.float32),
    )(partial, partial, WcT, bc2)


def kernel(x_user, x_tx, edge_index, edge_time, Wu, bu, Wt, bt, Wlin, att,
           time_beta, Wc, bc):
    H = att.shape[1]
    C = att.shape[2] // 2
    n_nodes = x_tx.shape[0]
    n_edges = edge_index.shape[1]

    Wx = Wt.T @ Wlin.T
    bx = bt @ Wlin.T
    att_i = att[0, :, :C]
    att_j = att[0, :, C:]
    eye = jnp.eye(H, dtype=jnp.float32)
    A_i = (att_i[:, :, None] * eye[:, None, :]).reshape(H * C, H)
    A_j = (att_j[:, :, None] * eye[:, None, :]).reshape(H * C, H)
    W1 = jnp.concatenate([Wx, Wx @ A_j, jnp.zeros((32, XAUG_D - 68))], axis=1)
    b1 = jnp.concatenate([bx, bx @ A_j, jnp.zeros(XAUG_D - 68)])[None]
    W2 = jnp.concatenate([Wx @ A_i, jnp.zeros((32, SI_D - 4))], axis=1)
    b2 = jnp.concatenate([bx @ A_i, jnp.zeros(SI_D - 4)])[None]

    xaug, si = _prep(x_tx, W1, b1, W2, b2)

    negbeta = jnp.full((16,), -jax.nn.softplus(time_beta), dtype=jnp.float32)

    n_chunks = n_edges // CH
    pad = (NW * SBC * NSB - n_chunks) * CH
    tbits = jax.lax.bitcast_convert_type(edge_time, jnp.int32)
    src_p = jnp.concatenate([edge_index[0], jnp.zeros((pad,), jnp.int32)])
    dst_p = jnp.concatenate(
        [edge_index[1], jnp.full((pad,), n_nodes, jnp.int32)]
    ).reshape(-1, CH)
    t_p = jnp.concatenate([tbits, jnp.zeros((pad,), jnp.int32)])

    edge_kernel = _make_edge_kernel(n_nodes, n_edges)
    partial = edge_kernel(src_p, dst_p, t_p, negbeta, xaug, si)

    return _tail(partial, Wc.T, bc[None], n_nodes)


# speedup vs baseline: 1.1754x; 1.0752x over previous
import functools

import jax
import jax.numpy as jnp
from jax import lax
from jax.experimental import pallas as pl
from jax.experimental.pallas import tpu as pltpu
from jax.experimental.pallas import tpu_sc as plsc

NC = 2
NS = 16
NW = NC * NS
CH = 128
HEADS = 4
XAUG_D = 80
SI_D = 16


def _prep_body(xtx_ref, w1_ref, b1_ref, w2_ref, b2_ref, xaug_ref, si_ref):
    x = xtx_ref[...]
    xaug_ref[...] = (
        jnp.dot(x, w1_ref[...], preferred_element_type=jnp.float32) + b1_ref[...]
    )
    si_ref[...] = (
        jnp.dot(x, w2_ref[...], preferred_element_type=jnp.float32) + b2_ref[...]
    )


def _prep(x_tx, W1, b1, W2, b2):
    n = x_tx.shape[0]
    blk = 5000
    return pl.pallas_call(
        _prep_body,
        grid=(n // blk,),
        in_specs=[
            pl.BlockSpec((blk, 32), lambda i: (i, 0)),
            pl.BlockSpec((32, XAUG_D), lambda i: (0, 0)),
            pl.BlockSpec((1, XAUG_D), lambda i: (0, 0)),
            pl.BlockSpec((32, SI_D), lambda i: (0, 0)),
            pl.BlockSpec((1, SI_D), lambda i: (0, 0)),
        ],
        out_specs=[
            pl.BlockSpec((blk, XAUG_D), lambda i: (i, 0)),
            pl.BlockSpec((blk, SI_D), lambda i: (i, 0)),
        ],
        out_shape=[
            jax.ShapeDtypeStruct((n, XAUG_D), jnp.float32),
            jax.ShapeDtypeStruct((n, SI_D), jnp.float32),
        ],
    )(x_tx, W1, b1, W2, b2)


NB = 4
SBC = 28
NSB = 7


def _make_edge_kernel(n_nodes, n_edges):
    n_chunks = n_edges // CH
    cpw = SBC * NSB
    zrows = 200
    n_rchunks = n_nodes // zrows
    mesh = plsc.VectorSubcoreMesh(core_axis_name="c", subcore_axis_name="s")

    @functools.partial(
        pl.kernel,
        mesh=mesh,
        out_type=jax.ShapeDtypeStruct((NC * n_nodes, 16), jnp.float32),
        scratch_types=[
            pltpu.VMEM((SBC * CH,), jnp.int32),
            pltpu.VMEM((SBC, CH), jnp.int32),
            pltpu.VMEM((SBC * CH,), jnp.float32),
            pltpu.VMEM((NB, CH, XAUG_D), jnp.float32),
            pltpu.VMEM((NB, CH, SI_D), jnp.float32),
            pltpu.VMEM((NB, CH, 16), jnp.float32),
            pltpu.VMEM((zrows, 16), jnp.float32),
            pltpu.VMEM((16,), jnp.float32),
            pltpu.VMEM_SHARED((n_nodes + 8, 16), jnp.float32),
            pltpu.SemaphoreType.DMA,
            pltpu.SemaphoreType.DMA,
            pltpu.SemaphoreType.DMA,
            pltpu.SemaphoreType.DMA,
            pltpu.SemaphoreType.DMA,
            pltpu.SemaphoreType.DMA,
            pltpu.SemaphoreType.DMA,
            pltpu.SemaphoreType.DMA,
            pltpu.SemaphoreType.DMA,
        ],
        compiler_params=pltpu.CompilerParams(use_tc_tiling_on_sc=False,
                                             needs_layout_passes=False),
    )
    def edge_kernel(src_hbm, dst_hbm, t_hbm, negbeta_hbm, xaug_hbm, si_hbm,
                    out_hbm, isrc, idst, itim, xrows, sirows, msg, zbuf,
                    nb_v, acc, sem_i, sem_g0, sem_g1, sem_g2, sem_g3,
                    sem_s0, sem_s1, sem_s2, sem_s3):
        sem_g = [sem_g0, sem_g1, sem_g2, sem_g3]
        sem_s = [sem_s0, sem_s1, sem_s2, sem_s3]
        cid = lax.axis_index("c")
        sid = lax.axis_index("s")
        wid = sid * NC + cid
        start = wid * cpw

        def issue_idx(s):
            e0 = (start + s * SBC) * CH
            pltpu.async_copy(src_hbm.at[pl.ds(e0, SBC * CH)], isrc, sem_i)
            pltpu.async_copy(dst_hbm.at[pl.ds(start + s * SBC, SBC)], idst,
                             sem_i)
            pltpu.async_copy(t_hbm.at[pl.ds(e0, SBC * CH)], itim, sem_i)

        def wait_idx(s):
            e0 = (start + s * SBC) * CH
            pltpu.make_async_copy(src_hbm.at[pl.ds(e0, SBC * CH)], isrc,
                                  sem_i).wait()
            pltpu.make_async_copy(dst_hbm.at[pl.ds(start + s * SBC, SBC)],
                                  idst, sem_i).wait()
            pltpu.make_async_copy(t_hbm.at[pl.ds(e0, SBC * CH)], itim,
                                  sem_i).wait()

        issue_idx(0)
        pltpu.sync_copy(negbeta_hbm, nb_v)

        def zrow_body(i, carry):
            zbuf[i, pl.ds(0, 16)] = jnp.zeros((16,), jnp.float32)
            return carry

        lax.fori_loop(0, zrows, zrow_body, 0)
        n_my_rchunks = (n_rchunks - sid + NS - 1) // NS

        def zchunk_body(j, carry):
            r0 = (sid + j * NS) * zrows
            pltpu.sync_copy(zbuf, acc.at[pl.ds(r0, zrows)])
            return carry

        lax.fori_loop(0, n_my_rchunks, zchunk_body, 0)
        plsc.subcore_barrier()

        nbvec = nb_v[pl.ds(0, 16)]
        lanes = lax.iota(jnp.int32, 16)
        zl = lanes * 0

        def issue_gather(j, b):
            pltpu.async_copy(xaug_hbm.at[isrc.at[pl.ds(j * CH, CH)]],
                             xrows.at[b], sem_g[b])
            pltpu.async_copy(si_hbm.at[idst.at[j]], sirows.at[b],
                             sem_g[b])

        def wait_gather(j, b):
            pltpu.make_async_copy(xaug_hbm.at[isrc.at[pl.ds(j * CH, CH)]],
                                  xrows.at[b], sem_g[b]).wait()
            pltpu.make_async_copy(si_hbm.at[idst.at[j]],
                                  sirows.at[b], sem_g[b]).wait()

        def compute_chunk(j, b):
            def group_body(g, gcarry):
                e0 = g * 16
                eidx = lanes + e0
                t = itim[pl.ds(j * CH + e0, 16)]
                tw = jnp.exp(t * nbvec)
                alphas = []
                for h in range(HEADS):
                    col = zl + h
                    si_h = plsc.load_gather(sirows.at[b], [eidx, col])
                    sj_h = plsc.load_gather(xrows.at[b], [eidx, col + 64])
                    a = si_h + sj_h
                    a = jnp.maximum(a, 0.2 * a) * tw
                    alphas.append(1.0 / (1.0 + jnp.exp(-a)))
                for lane in range(16):
                    e = e0 + lane
                    m = (xrows[b, e, pl.ds(0, 16)] * alphas[0][lane]
                         + xrows[b, e, pl.ds(16, 16)] * alphas[1][lane]
                         + xrows[b, e, pl.ds(32, 16)] * alphas[2][lane]
                         + xrows[b, e, pl.ds(48, 16)] * alphas[3][lane])
                    msg[b, e, pl.ds(0, 16)] = m
                return gcarry

            lax.fori_loop(0, CH // 16, group_body, 0)

        def wait_scatter(b):
            pltpu.make_async_copy(msg.at[b], acc.at[idst.at[b]],
                                  sem_s[b]).wait()

        def sb_body(s, carry):
            @pl.when(s > 0)
            def _():
                for b in range(NB):
                    wait_scatter(b)
                issue_idx(s)

            wait_idx(s)

            for b in range(NB):
                issue_gather(b, b)

            def q_body(q, qcarry):
                for b in range(NB):
                    j = q * NB + b
                    wait_gather(j, b)

                    @pl.when(q > 0)
                    def _():
                        wait_scatter(b)

                    compute_chunk(j, b)

                    @pl.when(j + NB < SBC)
                    def _():
                        issue_gather(j + NB, b)

                    pltpu.async_copy(msg.at[b], acc.at[idst.at[j]],
                                     sem_s[b], add=True)
                return qcarry

            lax.fori_loop(0, SBC // NB, q_body, 0)
            return carry

        lax.fori_loop(0, NSB, sb_body, 0)
        for b in range(NB):
            wait_scatter(b)

        plsc.subcore_barrier()

        def ochunk_body(j, carry):
            r0 = (sid + j * NS) * zrows
            pltpu.sync_copy(acc.at[pl.ds(r0, zrows)],
                            out_hbm.at[pl.ds(cid * n_nodes + r0, zrows)])
            return carry

        lax.fori_loop(0, n_my_rchunks, ochunk_body, 0)

    return edge_kernel


def _tail_body(p0_ref, p1_ref, wc_ref, bc_ref, out_ref):
    h = 0.25 * (p0_ref[...] + p1_ref[...])
    h = jnp.where(h > 0, h, jnp.exp(h) - 1.0)
    out_ref[...] = (
        jnp.dot(h, wc_ref[...], preferred_element_type=jnp.float32) + bc_ref[...]
    )


def _tail(partial, WcT, bc2, n_nodes):
    blk = 5000
    nb = n_nodes // blk
    out_d = WcT.shape[1]
    return pl.pallas_call(
        _tail_body,
        grid=(nb,),
        in_specs=[
            pl.BlockSpec((blk, 16), lambda i: (i, 0)),
            pl.BlockSpec((blk, 16), lambda i, nb=nb: (nb + i, 0)),
            pl.BlockSpec((16, out_d), lambda i: (0, 0)),
            pl.BlockSpec((1, out_d), lambda i: (0, 0)),
        ],
        out_specs=pl.BlockSpec((blk, out_d), lambda i: (i, 0)),
        out_shape=jax.ShapeDtypeStruct((n_nodes, out_d), jnp.float32),
    )(partial, partial, WcT, bc2)


def kernel(x_user, x_tx, edge_index, edge_time, Wu, bu, Wt, bt, Wlin, att,
           time_beta, Wc, bc):
    H = att.shape[1]
    C = att.shape[2] // 2
    n_nodes = x_tx.shape[0]
    n_edges = edge_index.shape[1]

    Wx = Wt.T @ Wlin.T
    bx = bt @ Wlin.T
    att_i = att[0, :, :C]
    att_j = att[0, :, C:]
    eye = jnp.eye(H, dtype=jnp.float32)
    A_i = (att_i[:, :, None] * eye[:, None, :]).reshape(H * C, H)
    A_j = (att_j[:, :, None] * eye[:, None, :]).reshape(H * C, H)
    W1 = jnp.concatenate([Wx, Wx @ A_j, jnp.zeros((32, XAUG_D - 68))], axis=1)
    b1 = jnp.concatenate([bx, bx @ A_j, jnp.zeros(XAUG_D - 68)])[None]
    W2 = jnp.concatenate([Wx @ A_i, jnp.zeros((32, SI_D - 4))], axis=1)
    b2 = jnp.concatenate([bx @ A_i, jnp.zeros(SI_D - 4)])[None]

    xaug, si = _prep(x_tx, W1, b1, W2, b2)

    negbeta = jnp.full((16,), -jax.nn.softplus(time_beta), dtype=jnp.float32)

    n_chunks = n_edges // CH
    pad = (NW * SBC * NSB - n_chunks) * CH
    spread = jnp.arange(pad, dtype=jnp.int32)
    src_p = jnp.concatenate([edge_index[0], spread % n_nodes])
    dst_p = jnp.concatenate(
        [edge_index[1], n_nodes + (spread & 7)]
    ).reshape(-1, CH)
    t_p = jnp.concatenate([edge_time, jnp.zeros((pad,), jnp.float32)])

    edge_kernel = _make_edge_kernel(n_nodes, n_edges)
    partial = edge_kernel(src_p, dst_p, t_p, negbeta, xaug, si)

    return _tail(partial, Wc.T, bc[None], n_nodes)


# speedup vs baseline: 1.4442x; 1.2287x over previous
import functools

import jax
import jax.numpy as jnp
from jax import lax
from jax.experimental import pallas as pl
from jax.experimental.pallas import tpu as pltpu
from jax.experimental.pallas import tpu_sc as plsc

NC = 2
NS = 16
NW = NC * NS
CH = 128
HEADS = 4
XAUG_D = 80
SI_D = 16


def _prep_body(xtx_ref, w1_ref, b1_ref, w2_ref, b2_ref, xaug_ref, si_ref):
    x = xtx_ref[...]
    xaug_ref[...] = (
        jnp.dot(x, w1_ref[...], preferred_element_type=jnp.float32) + b1_ref[...]
    )
    si_ref[...] = (
        jnp.dot(x, w2_ref[...], preferred_element_type=jnp.float32) + b2_ref[...]
    )


def _prep(x_tx, W1, b1, W2, b2):
    n = x_tx.shape[0]
    blk = 5000
    return pl.pallas_call(
        _prep_body,
        grid=(n // blk,),
        in_specs=[
            pl.BlockSpec((blk, 32), lambda i: (i, 0)),
            pl.BlockSpec((32, XAUG_D), lambda i: (0, 0)),
            pl.BlockSpec((1, XAUG_D), lambda i: (0, 0)),
            pl.BlockSpec((32, SI_D), lambda i: (0, 0)),
            pl.BlockSpec((1, SI_D), lambda i: (0, 0)),
        ],
        out_specs=[
            pl.BlockSpec((blk, XAUG_D), lambda i: (i, 0)),
            pl.BlockSpec((blk, SI_D), lambda i: (i, 0)),
        ],
        out_shape=[
            jax.ShapeDtypeStruct((n, XAUG_D), jnp.float32),
            jax.ShapeDtypeStruct((n, SI_D), jnp.float32),
        ],
    )(x_tx, W1, b1, W2, b2)


NB = 4
SBC = 28
NSB = 7


def _make_edge_kernel(n_nodes, n_edges):
    n_chunks = n_edges // CH
    cpw = SBC * NSB
    zrows = 200
    n_rchunks = n_nodes // zrows
    mesh = plsc.VectorSubcoreMesh(core_axis_name="c", subcore_axis_name="s")

    @functools.partial(
        pl.kernel,
        mesh=mesh,
        out_type=jax.ShapeDtypeStruct((NC * n_nodes, 16), jnp.float32),
        scratch_types=[
            pltpu.VMEM((SBC * CH,), jnp.int32),
            pltpu.VMEM((SBC, CH), jnp.int32),
            pltpu.VMEM((SBC * CH,), jnp.float32),
            pltpu.VMEM((NB, CH, XAUG_D), jnp.float32),
            pltpu.VMEM((NB, CH, SI_D), jnp.float32),
            pltpu.VMEM((NB, CH, 16), jnp.float32),
            pltpu.VMEM((zrows, 16), jnp.float32),
            pltpu.VMEM((16,), jnp.float32),
            pltpu.VMEM_SHARED((n_nodes + 8, 16), jnp.float32),
            pltpu.SemaphoreType.DMA,
            pltpu.SemaphoreType.DMA,
            pltpu.SemaphoreType.DMA,
            pltpu.SemaphoreType.DMA,
            pltpu.SemaphoreType.DMA,
            pltpu.SemaphoreType.DMA,
            pltpu.SemaphoreType.DMA,
            pltpu.SemaphoreType.DMA,
            pltpu.SemaphoreType.DMA,
        ],
        compiler_params=pltpu.CompilerParams(use_tc_tiling_on_sc=False,
                                             needs_layout_passes=False),
    )
    def edge_kernel(src_hbm, dst_hbm, t_hbm, negbeta_hbm, xaug_hbm, si_hbm,
                    out_hbm, isrc, idst, itim, xrows, sirows, msg, zbuf,
                    nb_v, acc, sem_i, sem_g0, sem_g1, sem_g2, sem_g3,
                    sem_s0, sem_s1, sem_s2, sem_s3):
        sem_g = [sem_g0, sem_g1, sem_g2, sem_g3]
        sem_s = [sem_s0, sem_s1, sem_s2, sem_s3]
        cid = lax.axis_index("c")
        sid = lax.axis_index("s")
        wid = sid * NC + cid
        start = wid * cpw

        def issue_idx(s):
            e0 = (start + s * SBC) * CH
            pltpu.async_copy(src_hbm.at[pl.ds(e0, SBC * CH)], isrc, sem_i)
            pltpu.async_copy(dst_hbm.at[pl.ds(start + s * SBC, SBC)], idst,
                             sem_i)
            pltpu.async_copy(t_hbm.at[pl.ds(e0, SBC * CH)], itim, sem_i)

        def wait_idx(s):
            e0 = (start + s * SBC) * CH
            pltpu.make_async_copy(src_hbm.at[pl.ds(e0, SBC * CH)], isrc,
                                  sem_i).wait()
            pltpu.make_async_copy(dst_hbm.at[pl.ds(start + s * SBC, SBC)],
                                  idst, sem_i).wait()
            pltpu.make_async_copy(t_hbm.at[pl.ds(e0, SBC * CH)], itim,
                                  sem_i).wait()

        issue_idx(0)
        pltpu.sync_copy(negbeta_hbm, nb_v)

        def zrow_body(i, carry):
            zbuf[i, pl.ds(0, 16)] = jnp.zeros((16,), jnp.float32)
            return carry

        lax.fori_loop(0, zrows, zrow_body, 0)
        n_my_rchunks = (n_rchunks - sid + NS - 1) // NS

        def zchunk_body(j, carry):
            r0 = (sid + j * NS) * zrows
            pltpu.sync_copy(zbuf, acc.at[pl.ds(r0, zrows)])
            return carry

        lax.fori_loop(0, n_my_rchunks, zchunk_body, 0)
        plsc.subcore_barrier()

        nbvec = nb_v[pl.ds(0, 16)]
        lanes = lax.iota(jnp.int32, 16)
        zl = lanes * 0

        def issue_gather(j, b):
            pltpu.async_copy(xaug_hbm.at[isrc.at[pl.ds(j * CH, CH)]],
                             xrows.at[b], sem_g[b])
            pltpu.async_copy(si_hbm.at[idst.at[j]], sirows.at[b],
                             sem_g[b])

        def wait_gather(j, b):
            pltpu.make_async_copy(xaug_hbm.at[isrc.at[pl.ds(j * CH, CH)]],
                                  xrows.at[b], sem_g[b]).wait()
            pltpu.make_async_copy(si_hbm.at[idst.at[j]],
                                  sirows.at[b], sem_g[b]).wait()

        def compute_chunk(j, b):
            @plsc.parallel_loop(0, CH // 16, step=1)
            def group_body(g):
                e0 = g * 16
                eidx = lanes + e0
                t = itim[pl.ds(j * CH + e0, 16)]
                tw = jnp.exp(t * nbvec)
                alphas = []
                for h in range(HEADS):
                    col = zl + h
                    si_h = plsc.load_gather(sirows.at[b], [eidx, col])
                    sj_h = plsc.load_gather(xrows.at[b], [eidx, col + 64])
                    a = si_h + sj_h
                    a = jnp.maximum(a, 0.2 * a) * tw
                    alphas.append(1.0 / (1.0 + jnp.exp(-a)))
                for lane in range(16):
                    e = e0 + lane
                    m01 = (xrows[b, e, pl.ds(0, 16)] * alphas[0][lane]
                           + xrows[b, e, pl.ds(16, 16)] * alphas[1][lane])
                    m23 = (xrows[b, e, pl.ds(32, 16)] * alphas[2][lane]
                           + xrows[b, e, pl.ds(48, 16)] * alphas[3][lane])
                    msg[b, e, pl.ds(0, 16)] = m01 + m23

        def wait_scatter(b):
            pltpu.make_async_copy(msg.at[b], acc.at[idst.at[b]],
                                  sem_s[b]).wait()

        def sb_body(s, carry):
            @pl.when(s > 0)
            def _():
                for b in range(NB):
                    wait_scatter(b)
                issue_idx(s)

            wait_idx(s)

            for b in range(NB):
                issue_gather(b, b)

            def q_body(q, qcarry):
                for b in range(NB):
                    j = q * NB + b
                    wait_gather(j, b)

                    @pl.when(q > 0)
                    def _():
                        wait_scatter(b)

                    compute_chunk(j, b)

                    @pl.when(j + NB < SBC)
                    def _():
                        issue_gather(j + NB, b)

                    pltpu.async_copy(msg.at[b], acc.at[idst.at[j]],
                                     sem_s[b], add=True)
                return qcarry

            lax.fori_loop(0, SBC // NB, q_body, 0)
            return carry

        lax.fori_loop(0, NSB, sb_body, 0)
        for b in range(NB):
            wait_scatter(b)

        plsc.subcore_barrier()

        def ochunk_body(j, carry):
            r0 = (sid + j * NS) * zrows
            pltpu.sync_copy(acc.at[pl.ds(r0, zrows)],
                            out_hbm.at[pl.ds(cid * n_nodes + r0, zrows)])
            return carry

        lax.fori_loop(0, n_my_rchunks, ochunk_body, 0)

    return edge_kernel


def _tail_body(p0_ref, p1_ref, wc_ref, bc_ref, out_ref):
    h = 0.25 * (p0_ref[...] + p1_ref[...])
    h = jnp.where(h > 0, h, jnp.exp(h) - 1.0)
    out_ref[...] = (
        jnp.dot(h, wc_ref[...], preferred_element_type=jnp.float32) + bc_ref[...]
    )


def _tail(partial, WcT, bc2, n_nodes):
    blk = 5000
    nb = n_nodes // blk
    out_d = WcT.shape[1]
    return pl.pallas_call(
        _tail_body,
        grid=(nb,),
        in_specs=[
            pl.BlockSpec((blk, 16), lambda i: (i, 0)),
            pl.BlockSpec((blk, 16), lambda i, nb=nb: (nb + i, 0)),
            pl.BlockSpec((16, out_d), lambda i: (0, 0)),
            pl.BlockSpec((1, out_d), lambda i: (0, 0)),
        ],
        out_specs=pl.BlockSpec((blk, out_d), lambda i: (i, 0)),
        out_shape=jax.ShapeDtypeStruct((n_nodes, out_d), jnp.float32),
    )(partial, partial, WcT, bc2)


def kernel(x_user, x_tx, edge_index, edge_time, Wu, bu, Wt, bt, Wlin, att,
           time_beta, Wc, bc):
    H = att.shape[1]
    C = att.shape[2] // 2
    n_nodes = x_tx.shape[0]
    n_edges = edge_index.shape[1]

    Wx = Wt.T @ Wlin.T
    bx = bt @ Wlin.T
    att_i = att[0, :, :C]
    att_j = att[0, :, C:]
    eye = jnp.eye(H, dtype=jnp.float32)
    A_i = (att_i[:, :, None] * eye[:, None, :]).reshape(H * C, H)
    A_j = (att_j[:, :, None] * eye[:, None, :]).reshape(H * C, H)
    W1 = jnp.concatenate([Wx, Wx @ A_j, jnp.zeros((32, XAUG_D - 68))], axis=1)
    b1 = jnp.concatenate([bx, bx @ A_j, jnp.zeros(XAUG_D - 68)])[None]
    W2 = jnp.concatenate([Wx @ A_i, jnp.zeros((32, SI_D - 4))], axis=1)
    b2 = jnp.concatenate([bx @ A_i, jnp.zeros(SI_D - 4)])[None]

    xaug, si = _prep(x_tx, W1, b1, W2, b2)

    negbeta = jnp.full((16,), -jax.nn.softplus(time_beta), dtype=jnp.float32)

    n_chunks = n_edges // CH
    pad = (NW * SBC * NSB - n_chunks) * CH
    spread = jnp.arange(pad, dtype=jnp.int32)
    src_p = jnp.concatenate([edge_index[0], spread % n_nodes])
    dst_p = jnp.concatenate(
        [edge_index[1], n_nodes + (spread & 7)]
    ).reshape(-1, CH)
    t_p = jnp.concatenate([edge_time, jnp.zeros((pad,), jnp.float32)])

    edge_kernel = _make_edge_kernel(n_nodes, n_edges)
    partial = edge_kernel(src_p, dst_p, t_p, negbeta, xaug, si)

    return _tail(partial, Wc.T, bc[None], n_nodes)
